# baseline (speedup 1.0000x reference)
import functools

import jax
import jax.numpy as jnp
from jax import lax
from jax.experimental import pallas as pl
from jax.experimental.pallas import tpu as pltpu
from jax.experimental.pallas import tpu_sc as plsc

D = 128
S = 16384
NC = 2
NS = 16
NW = NC * NS
C = 128
H = C // 2
TOK_PER_W = S // NW
NCHUNK = TOK_PER_W // C
NT = 3
NBUF = 2
L = 16


def _sc_body(ffe_hbm, eid_tab_hbm, dim_tab_hbm,
             fld_hbm, fam_hbm, ety_hbm, eid_hbm, dim_hbm, out_hbm,
             raw_v, idx_v, stage_v, ffe_sh, eid_sh, dim_sh, gsem, osem):
    cid = lax.axis_index("c")
    sid = lax.axis_index("s")
    wid = sid * NC + cid
    wbase = wid * TOK_PER_W
    ih = [pltpu.async_copy(ref.at[pl.ds(wbase, TOK_PER_W)], raw_v.at[t], gsem)
          for t, ref in enumerate((fld_hbm, fam_hbm, ety_hbm, eid_hbm,
                                   dim_hbm))]
    for tab_hbm, tab_sh, rows in ((ffe_hbm, ffe_sh, 1024 // NS),
                                  (eid_tab_hbm, eid_sh, 64 // NS),
                                  (dim_tab_hbm, dim_sh, 512 // NS)):
        tsl = pl.ds(sid * rows, rows)
        ih.append(pltpu.async_copy(tab_hbm.at[tsl], tab_sh.at[tsl], gsem))
    for h in ih:
        h.wait()
    plsc.subcore_barrier()

    def build_idx(ci):
        b = ci % NBUF
        for g in range(C // L):
            sl = pl.ds(ci * C + g * L, L)
            dsl = pl.ds(g * L, L)
            idx_v[b, dsl] = (raw_v[0, sl] * 32 + raw_v[1, sl] * 4
                             + raw_v[2, sl])

    def issue_gathers(ci):
        b = ci % NBUF
        csl = pl.ds(ci * C, C)
        return [
            pltpu.async_copy(ffe_sh.at[idx_v.at[b]], stage_v.at[b, 0], gsem),
            pltpu.async_copy(eid_sh.at[raw_v.at[3, csl]],
                             stage_v.at[b, 1], gsem),
            pltpu.async_copy(dim_sh.at[raw_v.at[4, csl]],
                             stage_v.at[b, 2], gsem),
        ]

    def accumulate(ci):
        b = ci % NBUF

        def row_body(r, carry):
            for g in range(D // L):
                sl = pl.ds(g * L, L)
                stage_v[b, 0, r, sl] = (stage_v[b, 0, r, sl]
                                        + stage_v[b, 1, r, sl]
                                        + stage_v[b, 2, r, sl])
            return carry

        lax.fori_loop(0, C, row_body, 0)

    gh = {}
    oh = {}

    def start(ci):
        if ci - NBUF in oh:
            oh.pop(ci - NBUF).wait()
        build_idx(ci)
        gh[ci] = issue_gathers(ci)

    def finish(cj):
        for h in gh.pop(cj):
            h.wait()
        accumulate(cj)
        oh[cj] = pltpu.async_copy(stage_v.at[cj % NBUF, 0],
                                  out_hbm.at[pl.ds(wbase + cj * C, C)], osem)

    depth = NBUF - 1
    for ci in range(depth):
        start(ci)
    for ci in range(depth, NCHUNK):
        start(ci)
        finish(ci - depth)
    for cj in range(NCHUNK - depth, NCHUNK):
        finish(cj)
    for h in oh.values():
        h.wait()


@functools.cache
def _sc_gather_sum_fn():
    return pl.kernel(
        _sc_body,
        out_type=jax.ShapeDtypeStruct((S, D), jnp.float32),
        mesh=plsc.VectorSubcoreMesh(core_axis_name="c", subcore_axis_name="s",
                                    num_cores=NC, num_subcores=NS),
        compiler_params=pltpu.CompilerParams(use_tc_tiling_on_sc=False),
        scratch_types=[
            pltpu.VMEM((5, TOK_PER_W), jnp.int32),
            pltpu.VMEM((NBUF, C), jnp.int32),
            pltpu.VMEM((NBUF, NT, C, D), jnp.float32),
            pltpu.VMEM_SHARED((1024, D), jnp.float32),
            pltpu.VMEM_SHARED((64, D), jnp.float32),
            pltpu.VMEM_SHARED((512, D), jnp.float32),
            pltpu.SemaphoreType.DMA,
            pltpu.SemaphoreType.DMA,
        ],
    )


def _mlp_body(v_ref, w1_ref, b1_ref, w2_ref, b2_ref, h_ref):
    v = v_ref[...]
    x = jnp.sign(v) * jnp.log1p(jnp.abs(v))
    h1 = x * w1_ref[...] + b1_ref[...]
    h1 = 0.5 * h1 * (1.0 + lax.erf(h1 * 0.7071067811865475))
    h_ref[...] = (jnp.dot(h1, w2_ref[...], preferred_element_type=jnp.float32)
                  + b2_ref[...])


def _ln_body(h_ref, g_ref, gm_ref, bt_ref, o_ref):
    h2 = h_ref[...] + g_ref[...]
    mean = jnp.mean(h2, axis=-1, keepdims=True)
    xc = h2 - mean
    var = jnp.mean(xc * xc, axis=-1, keepdims=True)
    o_ref[...] = xc * lax.rsqrt(var + 1e-5) * gm_ref[...] + bt_ref[...]


BT = 4096


def _tc_mlp(v2, W1, b1, W2, b2):
    vec = pl.BlockSpec((D,), lambda i: (0,))
    return pl.pallas_call(
        _mlp_body,
        grid=(S // BT,),
        in_specs=[
            pl.BlockSpec((BT, 1), lambda i: (i, 0)),
            pl.BlockSpec((1, D), lambda i: (0, 0)),
            vec,
            pl.BlockSpec((D, D), lambda i: (0, 0)),
            vec,
        ],
        out_specs=pl.BlockSpec((BT, D), lambda i: (i, 0)),
        out_shape=jax.ShapeDtypeStruct((S, D), jnp.float32),
    )(v2, W1, b1, W2, b2)


BT_LN = 2048


def _tc_add_ln(h, g, gamma, beta):
    vec = pl.BlockSpec((D,), lambda i: (0,))
    return pl.pallas_call(
        _ln_body,
        grid=(S // BT_LN,),
        in_specs=[
            pl.BlockSpec((BT_LN, D), lambda i: (i, 0)),
            pl.BlockSpec((BT_LN, D), lambda i: (i, 0)),
            vec, vec,
        ],
        out_specs=pl.BlockSpec((BT_LN, D), lambda i: (i, 0)),
        out_shape=jax.ShapeDtypeStruct((S, D), jnp.float32),
    )(h, g, gamma, beta)


def kernel(values, field_idx, family_idx, entity_type_idx, entity_id, dim_idx,
           field_emb, family_emb, entity_type_emb, entity_id_emb, dim_emb,
           W1, b1, W2, b2, gamma, beta):
    ffe = (field_emb[:, None, None, :] + family_emb[None, :, None, :]
           + entity_type_emb[None, None, :, :]).reshape(32 * 8 * 4, D)
    g = _sc_gather_sum_fn()(ffe, entity_id_emb, dim_emb,
                            field_idx, family_idx, entity_type_idx,
                            entity_id, dim_idx)
    h = _tc_mlp(values.reshape(S, 1), W1, b1, W2, b2)
    return _tc_add_ln(h, g, gamma, beta)


# speedup vs baseline: 1.0541x; 1.0541x over previous
import functools

import jax
import jax.numpy as jnp
from jax import lax
from jax.experimental import pallas as pl
from jax.experimental.pallas import tpu as pltpu
from jax.experimental.pallas import tpu_sc as plsc

D = 128
S = 16384
NC = 2
NS = 16
NW = NC * NS
C = 128
H = C // 2
TOK_PER_W = S // NW
NCHUNK = TOK_PER_W // C
NT = 3
NBUF = 2
L = 16


def _sc_body(ffe_hbm, eid_tab_hbm, dim_tab_hbm,
             fld_hbm, fam_hbm, ety_hbm, eid_hbm, dim_hbm, out_hbm,
             raw_v, idx_v, stage_v, ffe_sh, eid_sh, dim_sh, gsem, osem):
    cid = lax.axis_index("c")
    sid = lax.axis_index("s")
    wid = sid * NC + cid
    wbase = wid * TOK_PER_W
    ih = [pltpu.async_copy(ref.at[pl.ds(wbase, TOK_PER_W)], raw_v.at[t], gsem)
          for t, ref in enumerate((fld_hbm, fam_hbm, ety_hbm, eid_hbm,
                                   dim_hbm))]
    for tab_hbm, tab_sh, rows in ((ffe_hbm, ffe_sh, 1024 // NS),
                                  (eid_tab_hbm, eid_sh, 64 // NS),
                                  (dim_tab_hbm, dim_sh, 512 // NS)):
        tsl = pl.ds(sid * rows, rows)
        ih.append(pltpu.async_copy(tab_hbm.at[tsl], tab_sh.at[tsl], gsem))
    for h in ih:
        h.wait()
    plsc.subcore_barrier()

    def build_idx(ci):
        b = ci % NBUF
        for g in range(C // L):
            sl = pl.ds(ci * C + g * L, L)
            dsl = pl.ds(g * L, L)
            idx_v[b, dsl] = (raw_v[0, sl] * 32 + raw_v[1, sl] * 4
                             + raw_v[2, sl])

    def issue_gathers(ci):
        b = ci % NBUF
        csl = pl.ds(ci * C, C)
        return [
            pltpu.async_copy(ffe_sh.at[idx_v.at[b]], stage_v.at[b, 0], gsem),
            pltpu.async_copy(eid_sh.at[raw_v.at[3, csl]],
                             stage_v.at[b, 1], gsem),
            pltpu.async_copy(dim_sh.at[raw_v.at[4, csl]],
                             stage_v.at[b, 2], gsem),
        ]

    def accumulate(ci):
        b = ci % NBUF

        def row_body(r, carry):
            for g in range(D // L):
                sl = pl.ds(g * L, L)
                stage_v[b, 0, r, sl] = (stage_v[b, 0, r, sl]
                                        + stage_v[b, 1, r, sl]
                                        + stage_v[b, 2, r, sl])
            return carry

        lax.fori_loop(0, C, row_body, 0)

    gh = {}
    oh = {}

    def start(ci):
        if ci - NBUF in oh:
            oh.pop(ci - NBUF).wait()
        build_idx(ci)
        gh[ci] = issue_gathers(ci)

    def finish(cj):
        for h in gh.pop(cj):
            h.wait()
        accumulate(cj)
        oh[cj] = pltpu.async_copy(stage_v.at[cj % NBUF, 0],
                                  out_hbm.at[pl.ds(wbase + cj * C, C)], osem)

    depth = NBUF - 1
    for ci in range(depth):
        start(ci)
    for ci in range(depth, NCHUNK):
        start(ci)
        finish(ci - depth)
    for cj in range(NCHUNK - depth, NCHUNK):
        finish(cj)
    for h in oh.values():
        h.wait()


@functools.cache
def _sc_gather_sum_fn():
    return pl.kernel(
        _sc_body,
        out_type=jax.ShapeDtypeStruct((S, D), jnp.float32),
        mesh=plsc.VectorSubcoreMesh(core_axis_name="c", subcore_axis_name="s",
                                    num_cores=NC, num_subcores=NS),
        compiler_params=pltpu.CompilerParams(use_tc_tiling_on_sc=False),
        scratch_types=[
            pltpu.VMEM((5, TOK_PER_W), jnp.int32),
            pltpu.VMEM((NBUF, C), jnp.int32),
            pltpu.VMEM((NBUF, NT, C, D), jnp.float32),
            pltpu.VMEM_SHARED((1024, D), jnp.float32),
            pltpu.VMEM_SHARED((64, D), jnp.float32),
            pltpu.VMEM_SHARED((512, D), jnp.float32),
            pltpu.SemaphoreType.DMA,
            pltpu.SemaphoreType.DMA,
        ],
    )


def _mlp_body(v_ref, w1_ref, b1_ref, w2_ref, b2_ref, h_ref):
    v = v_ref[...]
    x = jnp.sign(v) * jnp.log1p(jnp.abs(v))
    h1 = x * w1_ref[...] + b1_ref[...]
    h1 = 0.5 * h1 * (1.0 + lax.erf(h1 * 0.7071067811865475))
    h_ref[...] = (jnp.dot(h1, w2_ref[...], preferred_element_type=jnp.float32)
                  + b2_ref[...])


def _ln_body(h_ref, g_ref, gm_ref, bt_ref, o_ref):
    h2 = h_ref[...] + g_ref[...]
    mean = jnp.mean(h2, axis=-1, keepdims=True)
    xc = h2 - mean
    var = jnp.mean(xc * xc, axis=-1, keepdims=True)
    o_ref[...] = xc * lax.rsqrt(var + 1e-5) * gm_ref[...] + bt_ref[...]


BT = 4096


def _tc_mlp(v2, W1, b1, W2, b2):
    vec = pl.BlockSpec((D,), lambda i: (0,))
    return pl.pallas_call(
        _mlp_body,
        grid=(S // BT,),
        in_specs=[
            pl.BlockSpec((BT, 1), lambda i: (i, 0)),
            pl.BlockSpec((1, D), lambda i: (0, 0)),
            vec,
            pl.BlockSpec((D, D), lambda i: (0, 0)),
            vec,
        ],
        out_specs=pl.BlockSpec((BT, D), lambda i: (i, 0)),
        out_shape=jax.ShapeDtypeStruct((S, D), jnp.float32),
    )(v2, W1, b1, W2, b2)


BT_LN = 8192


def _tc_add_ln(h, g, gamma, beta):
    vec = pl.BlockSpec((D,), lambda i: (0,))
    return pl.pallas_call(
        _ln_body,
        grid=(S // BT_LN,),
        in_specs=[
            pl.BlockSpec((BT_LN, D), lambda i: (i, 0)),
            pl.BlockSpec((BT_LN, D), lambda i: (i, 0)),
            vec, vec,
        ],
        out_specs=pl.BlockSpec((BT_LN, D), lambda i: (i, 0)),
        out_shape=jax.ShapeDtypeStruct((S, D), jnp.float32),
    )(h, g, gamma, beta)


def kernel(values, field_idx, family_idx, entity_type_idx, entity_id, dim_idx,
           field_emb, family_emb, entity_type_emb, entity_id_emb, dim_emb,
           W1, b1, W2, b2, gamma, beta):
    ffe = (field_emb[:, None, None, :] + family_emb[None, :, None, :]
           + entity_type_emb[None, None, :, :]).reshape(32 * 8 * 4, D)
    g = _sc_gather_sum_fn()(ffe, entity_id_emb, dim_emb,
                            field_idx, family_idx, entity_type_idx,
                            entity_id, dim_idx)
    h = _tc_mlp(values.reshape(S, 1), W1, b1, W2, b2)
    return _tc_add_ln(h, g, gamma, beta)
